# Initial kernel scaffold; baseline (speedup 1.0000x reference)
import jax
import jax.numpy as jnp
from jax.experimental import pallas as pl


def kernel(x, edge_index, edge_attr, W, b):
    raise NotImplementedError("write your pallas kernel here")


# speedup vs baseline: 41.2224x; 41.2224x over previous
import functools

import jax
import jax.numpy as jnp
from jax import lax
from jax.experimental import pallas as pl
from jax.experimental.pallas import tpu as pltpu
from jax.experimental.pallas import tpu_sc as plsc

NC = 2
NS = 16
NW = NC * NS
CH = 128
PAD_ROWS = 240


def _sc_mesh():
    return plsc.VectorSubcoreMesh(
        core_axis_name="c", subcore_axis_name="s", num_cores=NC, num_subcores=NS
    )


def _build_deg_kernel(n_pad, cpt):
    stripe = n_pad // NS

    @functools.partial(
        pl.kernel,
        out_type=jax.ShapeDtypeStruct((NC, n_pad), jnp.float32),
        mesh=_sc_mesh(),
        scratch_types=[
            pltpu.VMEM((cpt, CH), jnp.int32),
            pltpu.VMEM((CH,), jnp.float32),
            pltpu.VMEM_SHARED((n_pad,), jnp.float32),
        ],
    )
    def deg_kernel(dst_hbm, zeros_hbm, out_hbm, dst_v, ones_v, acc):
        cid = lax.axis_index("c")
        sid = lax.axis_index("s")
        wid = cid * NS + sid
        pltpu.sync_copy(dst_hbm.at[wid], dst_v)
        for k in range(CH // 16):
            ones_v[pl.ds(k * 16, 16)] = jnp.full((16,), 1.0, jnp.float32)
        pltpu.sync_copy(zeros_hbm, acc.at[pl.ds(sid * stripe, stripe)])
        plsc.subcore_barrier()

        @pl.loop(0, cpt)
        def _(j):
            pltpu.sync_copy(ones_v, acc.at[dst_v.at[j]], add=True)

        plsc.subcore_barrier()
        pltpu.sync_copy(
            acc.at[pl.ds(sid * stripe, stripe)],
            out_hbm.at[cid, pl.ds(sid * stripe, stripe)],
        )

    return deg_kernel


def _build_scatter_kernel(n_pad, cpt, d):
    stripe = n_pad // NS
    assert cpt % 4 == 0
    half = cpt // 2

    @functools.partial(
        pl.kernel,
        out_type=jax.ShapeDtypeStruct((NC, n_pad, d), jnp.float32),
        mesh=_sc_mesh(),
        scratch_types=[
            pltpu.VMEM((half, CH), jnp.int32),
            pltpu.VMEM((half, CH), jnp.int32),
            pltpu.VMEM((CH, d), jnp.float32),
            pltpu.VMEM((CH, d), jnp.float32),
            pltpu.VMEM_SHARED((n_pad, d), jnp.float32),
            pltpu.SemaphoreType.DMA,
            pltpu.SemaphoreType.DMA,
        ],
    )
    def scat_kernel(src_hbm, dst_hbm, h2_hbm, zeros_hbm, out_hbm,
                    src_v, dst_v, bufa, bufb, acc, sema, semb):
        cid = lax.axis_index("c")
        sid = lax.axis_index("s")
        wid = cid * NS + sid
        pltpu.sync_copy(zeros_hbm, acc.at[pl.ds(sid * stripe, stripe)])
        plsc.subcore_barrier()

        for hf in range(2):
            pltpu.sync_copy(src_hbm.at[wid, pl.ds(hf * half, half)], src_v)
            pltpu.sync_copy(dst_hbm.at[wid, pl.ds(hf * half, half)], dst_v)

            pltpu.async_copy(h2_hbm.at[src_v.at[0]], bufa, sema)

            @pl.loop(0, half, step=2)
            def _(j):
                pltpu.async_copy(h2_hbm.at[src_v.at[j + 1]], bufb, semb)
                pltpu.make_async_copy(h2_hbm.at[src_v.at[j]], bufa, sema).wait()
                pltpu.sync_copy(bufa, acc.at[dst_v.at[j]], add=True)

                @pl.when(j + 2 < half)
                def _():
                    pltpu.async_copy(h2_hbm.at[src_v.at[j + 2]], bufa, sema)

                pltpu.make_async_copy(h2_hbm.at[src_v.at[j + 1]], bufb, semb).wait()
                pltpu.sync_copy(bufb, acc.at[dst_v.at[j + 1]], add=True)

        plsc.subcore_barrier()
        pltpu.sync_copy(
            acc.at[pl.ds(sid * stripe, stripe)],
            out_hbm.at[cid, pl.ds(sid * stripe, stripe)],
        )

    return scat_kernel


def _matmul(x_pad, W, blk):
    n_pad, d_in = x_pad.shape
    d_out = W.shape[0]

    def body(x_ref, w_ref, h_ref):
        h_ref[...] = lax.dot_general(
            x_ref[...], w_ref[...], (((1,), (1,)), ((), ())),
            preferred_element_type=jnp.float32,
            precision=lax.Precision.HIGHEST,
        )

    return pl.pallas_call(
        body,
        grid=(n_pad // blk,),
        in_specs=[
            pl.BlockSpec((blk, d_in), lambda i: (i, 0)),
            pl.BlockSpec((d_out, d_in), lambda i: (0, 0)),
        ],
        out_specs=pl.BlockSpec((blk, d_out), lambda i: (i, 0)),
        out_shape=jax.ShapeDtypeStruct((n_pad, d_out), jnp.float32),
    )(x_pad, W)


def _scale(deg_partials_t, h, blk):
    n_pad, d = h.shape

    def body(dp_ref, h_ref, h2_ref, dis_ref):
        deg = dp_ref[:, 0:1] + dp_ref[:, 1:2] + 1.0
        dis = lax.rsqrt(deg)
        dis_ref[...] = dis
        h2_ref[...] = h_ref[...] * dis

    return pl.pallas_call(
        body,
        grid=(n_pad // blk,),
        in_specs=[
            pl.BlockSpec((blk, NC), lambda i: (i, 0)),
            pl.BlockSpec((blk, d), lambda i: (i, 0)),
        ],
        out_specs=[
            pl.BlockSpec((blk, d), lambda i: (i, 0)),
            pl.BlockSpec((blk, 1), lambda i: (i, 0)),
        ],
        out_shape=[
            jax.ShapeDtypeStruct((n_pad, d), jnp.float32),
            jax.ShapeDtypeStruct((n_pad, 1), jnp.float32),
        ],
    )(deg_partials_t, h)


def _epilogue(acc_partials, h2, dis, b2, blk):
    nc, n_pad, d = acc_partials.shape

    def body(ap_ref, h2_ref, dis_ref, b_ref, o_ref):
        s = ap_ref[0] + ap_ref[1] + h2_ref[...]
        o_ref[...] = s * dis_ref[...] + b_ref[...]

    return pl.pallas_call(
        body,
        grid=(n_pad // blk,),
        in_specs=[
            pl.BlockSpec((nc, blk, d), lambda i: (0, i, 0)),
            pl.BlockSpec((blk, d), lambda i: (i, 0)),
            pl.BlockSpec((blk, 1), lambda i: (i, 0)),
            pl.BlockSpec((1, d), lambda i: (0, 0)),
        ],
        out_specs=pl.BlockSpec((blk, d), lambda i: (i, 0)),
        out_shape=jax.ShapeDtypeStruct((n_pad, d), jnp.float32),
    )(acc_partials, h2, dis, b2)


def kernel(x, edge_index, edge_attr, W, b):
    n, d_in = x.shape
    d = W.shape[0]
    e = edge_index.shape[1]

    align = NS * 8
    n_pad = ((n + PAD_ROWS + align - 1) // align) * align
    cpt = -(-e // (NW * CH))
    cpt = ((cpt + 3) // 4) * 4
    e_pad = NW * cpt * CH
    stripe = n_pad // NS

    src = edge_index[0].astype(jnp.int32)
    dst = edge_index[1].astype(jnp.int32)
    npad_e = e_pad - e
    pad_i = jnp.arange(npad_e, dtype=jnp.int32)
    src_all = jnp.concatenate([src, pad_i % n]).reshape(NW, cpt, CH)
    dst_all = jnp.concatenate([dst, n + pad_i % (n_pad - n)]).reshape(NW, cpt, CH)

    zeros1 = jnp.zeros((stripe,), jnp.float32)
    zeros2 = jnp.zeros((stripe, d), jnp.float32)
    x_pad = jnp.pad(x, ((0, n_pad - n), (0, 0)))

    deg_partials = _build_deg_kernel(n_pad, cpt)(dst_all, zeros1)
    h = _matmul(x_pad, W, blk=1280)
    h2, dis = _scale(deg_partials.T, h, blk=1280)
    acc_partials = _build_scatter_kernel(n_pad, cpt, d)(src_all, dst_all, h2, zeros2)
    out = _epilogue(acc_partials, h2, dis, b.reshape(1, d), blk=1280)
    return out[:n]
